# Initial kernel scaffold; baseline (speedup 1.0000x reference)
import jax
import jax.numpy as jnp
from jax.experimental import pallas as pl


def kernel(x, edge_index, edge_weight, batch, W, b):
    raise NotImplementedError("write your pallas kernel here")


# speedup vs baseline: 6.9762x; 6.9762x over previous
import jax
import jax.numpy as jnp
from jax import lax
from jax.experimental import pallas as pl
from jax.experimental.pallas import tpu as pltpu
from jax.experimental.pallas import tpu_sc as plsc

N = 10000
E = 320000
D = 128
K = 64

NC = 2
NS = 16
NW = NC * NS
EPW = E // NW
CHUNK = 80
NCHUNK = EPW // CHUNK
ROWS_PT = N // NS
ZROWS = 125

_SELU_ALPHA = 1.6732632423543772
_SELU_SCALE = 1.0507009873554805


def _softmax_body(x_ref, w_ref, b_ref, s_ref, sbf_ref):
    logits = jnp.dot(x_ref[...].astype(jnp.bfloat16),
                     w_ref[...].astype(jnp.bfloat16),
                     preferred_element_type=jnp.float32) + b_ref[...]
    m = jnp.max(logits, axis=1, keepdims=True)
    e = jnp.exp(logits - m)
    s = e / jnp.sum(e, axis=1, keepdims=True)
    s_ref[...] = s
    sbf_ref[...] = s.astype(jnp.bfloat16).astype(jnp.float32)


def _assignments(x, W, b):
    nb = 1000
    return pl.pallas_call(
        _softmax_body,
        grid=(N // nb,),
        in_specs=[
            pl.BlockSpec((nb, D), lambda i: (i, 0)),
            pl.BlockSpec((D, K), lambda i: (0, 0)),
            pl.BlockSpec((1, K), lambda i: (0, 0)),
        ],
        out_specs=[
            pl.BlockSpec((nb, K), lambda i: (i, 0)),
            pl.BlockSpec((nb, K), lambda i: (i, 0)),
        ],
        out_shape=[
            jax.ShapeDtypeStruct((N, K), jnp.float32),
            jax.ShapeDtypeStruct((N, K), jnp.float32),
        ],
    )(x, W, b.reshape(1, K))


def _rne_bf16(v):
    u = plsc.bitcast(v, jnp.uint32)
    lsb = (u >> 16) & jnp.uint32(1)
    r = u + jnp.uint32(0x7FFF) + lsb
    return plsc.bitcast(r & jnp.uint32(0xFFFF0000), jnp.float32)


def _sc_body(s_hbm, sbf_hbm, src_hbm, dst_hbm, w_hbm, p_out, deg_out, tr_out,
             src_loc, dst_loc, w_loc, rows, rows_b, zbuf, deg_loc, tr_loc,
             p_spmem, sem, semb):
    cid = lax.axis_index("c")
    sid = lax.axis_index("s")
    wid = sid * NC + cid

    pltpu.sync_copy(src_hbm.at[wid], src_loc)
    pltpu.sync_copy(dst_hbm.at[wid], dst_loc)
    pltpu.sync_copy(w_hbm.at[wid], w_loc)

    zvec = jnp.zeros((16,), jnp.float32)

    def _zero_deg(i, _):
        deg_loc[pl.ds(i * 16, 16)] = zvec
        return 0
    lax.fori_loop(0, N // 16, _zero_deg, 0)

    def _zero_zbuf(r, _):
        for j in range(K // 16):
            zbuf[r, pl.ds(j * 16, 16)] = zvec
        return 0
    lax.fori_loop(0, ZROWS, _zero_zbuf, 0)

    def _zero_p(z, _):
        pltpu.sync_copy(zbuf, p_spmem.at[pl.ds(sid * ROWS_PT + z * ZROWS,
                                               ZROWS)])
        return 0
    lax.fori_loop(0, ROWS_PT // ZROWS, _zero_p, 0)
    plsc.subcore_barrier()

    tr_loc[...] = zvec

    def _chunk(c, _):
        ga = pltpu.async_copy(s_hbm.at[src_loc.at[c]], rows, sem)
        gb = pltpu.async_copy(sbf_hbm.at[dst_loc.at[c]], rows_b, semb)
        ga.wait()
        gb.wait()

        for t in range(CHUNK // 16):
            dvec = dst_loc[c, pl.ds(t * 16, 16)]
            wvec = w_loc[c, pl.ds(t * 16, 16)]
            plsc.addupdate_scatter(deg_loc, [dvec], wvec)

        cacc = jnp.zeros((16,), jnp.float32)
        for t in range(CHUNK // 16):
            wv16 = w_loc[c, pl.ds(t * 16, 16)]
            for e16 in range(16):
                ws = wv16[e16]
                e = t * 16 + e16
                for j in range(K // 16):
                    sl = pl.ds(j * 16, 16)
                    a = rows[e, sl] * ws
                    rows[e, sl] = a
                    cacc = cacc + _rne_bf16(a) * rows_b[e, sl]
        tr_loc[...] = tr_loc[...] + cacc

        pltpu.sync_copy(rows, p_spmem.at[dst_loc.at[c]], add=True)
        return 0

    lax.fori_loop(0, NCHUNK, _chunk, 0)
    plsc.subcore_barrier()

    pltpu.sync_copy(deg_loc, deg_out.at[wid])
    pltpu.sync_copy(tr_loc, tr_out.at[wid])
    pltpu.sync_copy(p_spmem.at[pl.ds(sid * ROWS_PT, ROWS_PT)],
                    p_out.at[cid, pl.ds(sid * ROWS_PT, ROWS_PT)])


def _sc_spmm(s, s_bf, src3, dst3, w3):
    mesh = plsc.VectorSubcoreMesh(core_axis_name="c", subcore_axis_name="s",
                                  num_cores=NC, num_subcores=NS)
    f = pl.kernel(
        _sc_body,
        out_type=[
            jax.ShapeDtypeStruct((NC, N, K), jnp.float32),
            jax.ShapeDtypeStruct((NW, N), jnp.float32),
            jax.ShapeDtypeStruct((NW, 16), jnp.float32),
        ],
        mesh=mesh,
        scratch_types=[
            pltpu.VMEM((NCHUNK, CHUNK), jnp.int32),
            pltpu.VMEM((NCHUNK, CHUNK), jnp.int32),
            pltpu.VMEM((NCHUNK, CHUNK), jnp.float32),
            pltpu.VMEM((CHUNK, K), jnp.float32),
            pltpu.VMEM((CHUNK, K), jnp.float32),
            pltpu.VMEM((ZROWS, K), jnp.float32),
            pltpu.VMEM((N,), jnp.float32),
            pltpu.VMEM((16,), jnp.float32),
            pltpu.VMEM_SHARED((N, K), jnp.float32),
            pltpu.SemaphoreType.DMA,
            pltpu.SemaphoreType.DMA,
        ],
        compiler_params=pltpu.CompilerParams(use_tc_tiling_on_sc=False,
                                             needs_layout_passes=False),
    )
    return f(s, s_bf, src3, dst3, w3)


def _finalize_body(x_ref, s_ref, sbf_ref, p_ref, deg_ref, tr_ref,
                   outx_ref, adj_ref, misc_ref,
                   acc_sx, acc_ss, acc_sp, acc_sd, acc_cs, acc_m2):
    i = pl.program_id(0)
    ng = pl.num_programs(0)

    @pl.when(i == 0)
    def _init():
        acc_sx[...] = jnp.zeros_like(acc_sx)
        acc_ss[...] = jnp.zeros_like(acc_ss)
        acc_sp[...] = jnp.zeros_like(acc_sp)
        acc_sd[...] = jnp.zeros_like(acc_sd)
        acc_cs[...] = jnp.zeros_like(acc_cs)
        acc_m2[0, 0] = 0.0

    s_blk = s_ref[...]
    x_blk = x_ref[...]
    p_blk = p_ref[0] + p_ref[1]
    deg_blk = jnp.sum(deg_ref[...], axis=1)
    deg_bf = deg_blk.astype(jnp.bfloat16).astype(jnp.float32)

    dn = (((0,), (0,)), ((), ()))
    hi = lax.Precision.HIGHEST
    acc_sx[...] += lax.dot_general(s_blk, x_blk, dn, precision=hi,
                                   preferred_element_type=jnp.float32)
    acc_ss[...] += lax.dot_general(s_blk, s_blk, dn, precision=hi,
                                   preferred_element_type=jnp.float32)
    acc_sp[...] += lax.dot_general(s_blk, p_blk, dn, precision=hi,
                                   preferred_element_type=jnp.float32)
    acc_sd[...] += jnp.sum(sbf_ref[...] * deg_bf[:, None], axis=0,
                           keepdims=True)
    acc_cs[...] += jnp.sum(s_blk, axis=0, keepdims=True)
    acc_m2[0, 0] += jnp.sum(deg_blk)

    @pl.when(i == ng - 1)
    def _final():
        sx = acc_sx[...]
        outx_ref[...] = _SELU_SCALE * jnp.where(
            sx > 0.0, sx, _SELU_ALPHA * (jnp.exp(jnp.minimum(sx, 0.0)) - 1.0))

        adj_t = acc_sp[...]
        sd = acc_sd[...]
        cs = acc_cs[...]
        cc = acc_ss[...]
        m2 = acc_m2[0, 0]

        r = lax.broadcasted_iota(jnp.int32, (K, K), 0)
        c = lax.broadcasted_iota(jnp.int32, (K, K), 1)
        eye = (r == c)

        tr = jnp.sum(tr_ref[...])
        spectral = -(tr - jnp.sum(sd * sd) / m2) / m2

        sqrt_k = jnp.sqrt(jnp.float32(K))
        cluster = jnp.sqrt(jnp.sum(cs * cs)) / jnp.float32(N) * sqrt_k - 1.0

        cc_norm = jnp.sqrt(jnp.sum(cc * cc))
        eye_f = jnp.where(eye, 1.0 / sqrt_k, 0.0)
        ortho = jnp.sqrt(jnp.sum((cc / cc_norm - eye_f) ** 2))

        adj_tm = jnp.where(eye, 0.0, adj_t)
        dd = jnp.sqrt(jnp.sum(adj_tm, axis=0, keepdims=True)) + 1e-12
        dd_outer = lax.dot_general(dd, dd, (((0,), (0,)), ((), ())),
                                   precision=lax.Precision.HIGHEST,
                                   preferred_element_type=jnp.float32)
        adj_tn = adj_tm / dd_outer
        eye_m = jnp.where(eye, 1.0, 0.0)
        adj_ref[...] = lax.dot_general(adj_tn, eye_m, dn,
                                       precision=lax.Precision.HIGHEST,
                                       preferred_element_type=jnp.float32)

        mi = lax.broadcasted_iota(jnp.int32, (8, 128), 0)
        mj = lax.broadcasted_iota(jnp.int32, (8, 128), 1)
        row0 = mi == 0
        misc_ref[...] = jnp.where(
            row0 & (mj == 0), spectral,
            jnp.where(row0 & (mj == 1), cluster,
                      jnp.where(row0 & (mj == 2), ortho, 0.0)))


def _finalize(x, s, s_bf, p2, deg2, tr2):
    nb = 1000
    return pl.pallas_call(
        _finalize_body,
        grid=(N // nb,),
        in_specs=[
            pl.BlockSpec((nb, D), lambda i: (i, 0)),
            pl.BlockSpec((nb, K), lambda i: (i, 0)),
            pl.BlockSpec((nb, K), lambda i: (i, 0)),
            pl.BlockSpec((NC, nb, K), lambda i: (0, i, 0)),
            pl.BlockSpec((nb, NW), lambda i: (i, 0)),
            pl.BlockSpec((NW, 16), lambda i: (0, 0)),
        ],
        out_specs=[
            pl.BlockSpec((K, D), lambda i: (0, 0)),
            pl.BlockSpec((K, K), lambda i: (0, 0)),
            pl.BlockSpec((8, 128), lambda i: (0, 0)),
        ],
        out_shape=[
            jax.ShapeDtypeStruct((K, D), jnp.float32),
            jax.ShapeDtypeStruct((K, K), jnp.float32),
            jax.ShapeDtypeStruct((8, 128), jnp.float32),
        ],
        scratch_shapes=[
            pltpu.VMEM((K, D), jnp.float32),
            pltpu.VMEM((K, K), jnp.float32),
            pltpu.VMEM((K, K), jnp.float32),
            pltpu.VMEM((1, K), jnp.float32),
            pltpu.VMEM((1, K), jnp.float32),
            pltpu.SMEM((1, 1), jnp.float32),
        ],
    )(x, s, s_bf, p2, deg2, tr2)


@jax.jit
def kernel(x, edge_index, edge_weight, batch, W, b):
    src3 = edge_index[0].astype(jnp.int32).reshape(NW, NCHUNK, CHUNK)
    dst3 = edge_index[1].astype(jnp.int32).reshape(NW, NCHUNK, CHUNK)
    w3 = edge_weight.astype(jnp.float32).reshape(NW, NCHUNK, CHUNK)

    s, s_bf = _assignments(x, W, b)
    p2, deg2, tr2 = _sc_spmm(s, s_bf, src3, dst3, w3)
    out_x, adj_n, misc = _finalize(x, s, s_bf, p2, deg2.T, tr2)

    return (out_x, adj_n, misc[0, 0], misc[0, 1], misc[0, 2])
